# baseline (speedup 1.0000x reference)
import functools

import jax
import jax.numpy as jnp
from jax import lax
from jax.experimental import pallas as pl
from jax.experimental.pallas import tpu as pltpu
from jax.experimental.pallas import tpu_sc as plsc

CAT = 26
NUMD = 50
XDIM = CAT + NUMD
VOCAB = 100000
EMB = 16
FDIM = 128
CAT_EMB = CAT * EMB


@functools.lru_cache(maxsize=None)
def _make_gather(n_tok: int):
    info = plsc.get_sparse_core_info()
    nw = info.num_cores * info.num_subcores
    tok_per_w = n_tok // nw
    assert tok_per_w * nw == n_tok
    tchunk = tok_per_w // 2
    mesh = plsc.VectorSubcoreMesh(core_axis_name="c", subcore_axis_name="s")

    @functools.partial(
        pl.kernel,
        mesh=mesh,
        compiler_params=pltpu.CompilerParams(use_tc_tiling_on_sc=False),
        out_type=jax.ShapeDtypeStruct((n_tok, CAT_EMB), jnp.float32),
        scratch_types=[
            pltpu.VMEM((tchunk,), jnp.int32),
            pltpu.VMEM((tchunk, EMB), jnp.float32),
            pltpu.SemaphoreType.DMA,
        ],
    )
    def gather_k(idx_hbm, table_hbm, out_hbm, idx_v, rows_v, sem):
        wid = lax.axis_index("s") * info.num_cores + lax.axis_index("c")
        w_base = wid * tok_per_w

        def body(step, carry):
            c = step // 2
            tok = w_base + (step % 2) * tchunk
            pltpu.sync_copy(idx_hbm.at[c, pl.ds(tok, tchunk)], idx_v)
            pltpu.async_copy(table_hbm.at[idx_v], rows_v, sem).wait()
            pltpu.sync_copy(
                rows_v, out_hbm.at[pl.ds(tok, tchunk), pl.ds(c * EMB, EMB)]
            )
            return carry

        lax.fori_loop(0, 2 * CAT, body, 0)

    return gather_k


VP = 100352
_TBLK = 1024


def _transpose_tables(tables_t):
    nblk = VP // _TBLK

    def body(a_ref, out_ref):
        j = pl.program_id(1)
        out_ref[...] = jnp.transpose(
            a_ref[0, :, pl.ds(j * _TBLK, _TBLK)], (1, 0)
        )

    return pl.pallas_call(
        body,
        grid=(CAT, nblk),
        in_specs=[pl.BlockSpec((1, EMB, VOCAB), lambda i, j: (i, 0, 0))],
        out_specs=pl.BlockSpec((_TBLK, EMB), lambda i, j: (i * nblk + j, 0)),
        out_shape=jax.ShapeDtypeStruct((CAT * VP, EMB), jnp.float32),
    )(tables_t)


_TM = 2048


def _matmul(cat_emb, x2, Wn_pad, bn, Wocat, Wonum, bo):
    n = cat_emb.shape[0]

    def body(cat_ref, x_ref, wn_ref, bn_ref, wc_ref, wo_ref, bo_ref, out_ref):
        num_emb = (
            jnp.dot(x_ref[...], wn_ref[...], preferred_element_type=jnp.float32)
            + bn_ref[...]
        )
        acc = jnp.dot(cat_ref[...], wc_ref[...], preferred_element_type=jnp.float32)
        acc = acc + jnp.dot(num_emb, wo_ref[...], preferred_element_type=jnp.float32)
        out_ref[...] = acc + bo_ref[...]

    return pl.pallas_call(
        body,
        grid=(n // _TM,),
        in_specs=[
            pl.BlockSpec((_TM, CAT_EMB), lambda i: (i, 0)),
            pl.BlockSpec((_TM, XDIM), lambda i: (i, 0)),
            pl.BlockSpec((XDIM, NUMD), lambda i: (0, 0)),
            pl.BlockSpec((1, NUMD), lambda i: (0, 0)),
            pl.BlockSpec((CAT_EMB, FDIM), lambda i: (0, 0)),
            pl.BlockSpec((NUMD, FDIM), lambda i: (0, 0)),
            pl.BlockSpec((1, FDIM), lambda i: (0, 0)),
        ],
        out_specs=pl.BlockSpec((_TM, FDIM), lambda i: (i, 0)),
        out_shape=jax.ShapeDtypeStruct((n, FDIM), jnp.float32),
    )(cat_emb, x2, Wn_pad, bn, Wocat, Wonum, bo)


def kernel(x, tables, Wn, bn, Wo, bo):
    b, s, _ = x.shape
    n = b * s
    xt = jnp.swapaxes(x, 0, 1)
    idx_cm = jnp.transpose(xt[..., :CAT].astype(jnp.int32), (2, 0, 1)).reshape(
        CAT, n
    ) + (jnp.arange(CAT, dtype=jnp.int32) * VP)[:, None]
    tbl_lin = _transpose_tables(jnp.swapaxes(tables, 1, 2))
    cat_emb = _make_gather(n)(idx_cm, tbl_lin)
    x2 = xt.reshape(n, XDIM)
    Wn_pad = jnp.concatenate([jnp.zeros((CAT, NUMD), jnp.float32), Wn], axis=0)
    out2 = _matmul(
        cat_emb,
        x2,
        Wn_pad,
        bn.reshape(1, NUMD),
        Wo[:CAT_EMB],
        Wo[CAT_EMB:],
        bo.reshape(1, FDIM),
    )
    return jnp.swapaxes(out2.reshape(s, b, FDIM), 0, 1)


# speedup vs baseline: 1.3650x; 1.3650x over previous
import functools

import jax
import jax.numpy as jnp
from jax import lax
from jax.experimental import pallas as pl
from jax.experimental.pallas import tpu as pltpu
from jax.experimental.pallas import tpu_sc as plsc

CAT = 26
NUMD = 50
XDIM = CAT + NUMD
VOCAB = 100000
EMB = 16
FDIM = 128
CAT_EMB = CAT * EMB
CPAD = 512
VP = 100352
_TBLK = 1024


def _transpose_tables(tables_t):
    nblk = VP // _TBLK

    def body(a_ref, out_ref):
        j = pl.program_id(1)
        for d in range(8):
            s = a_ref[0, :, pl.ds(j * _TBLK + d * 128, 128)]
            out_ref[:, d * EMB : (d + 1) * EMB] = jnp.transpose(s, (1, 0))

    return pl.pallas_call(
        body,
        grid=(CAT, nblk),
        in_specs=[pl.BlockSpec((1, EMB, VOCAB), lambda i, j: (i, 0, 0))],
        out_specs=pl.BlockSpec(
            (_TBLK // 8, 128), lambda i, j: (i * nblk + j, 0)
        ),
        out_shape=jax.ShapeDtypeStruct((CAT * VP // 8, 128), jnp.float32),
    )(tables_t)


@functools.lru_cache(maxsize=None)
def _make_gather(n_tok: int):
    info = plsc.get_sparse_core_info()
    nw = info.num_cores * info.num_subcores
    tok_per_w = n_tok // nw
    assert tok_per_w * nw == n_tok
    tchunk = tok_per_w // 2
    mesh = plsc.VectorSubcoreMesh(core_axis_name="c", subcore_axis_name="s")

    @functools.partial(
        pl.kernel,
        mesh=mesh,
        compiler_params=pltpu.CompilerParams(use_tc_tiling_on_sc=False),
        out_type=jax.ShapeDtypeStruct((n_tok, CPAD), jnp.float32),
        scratch_types=[
            pltpu.VMEM((tchunk,), jnp.int32),
            pltpu.VMEM((tchunk, EMB), jnp.float32),
            pltpu.SemaphoreType.DMA,
        ],
    )
    def gather_k(idx_hbm, table_hbm, out_hbm, idx_v, rows_v, sem):
        wid = lax.axis_index("s") * info.num_cores + lax.axis_index("c")
        w_base = wid * tok_per_w

        def body(step, carry):
            c = step // 2
            tok = w_base + (step % 2) * tchunk
            pltpu.sync_copy(idx_hbm.at[c, pl.ds(tok, tchunk)], idx_v)
            pltpu.async_copy(table_hbm.at[idx_v], rows_v, sem).wait()
            pltpu.sync_copy(
                rows_v, out_hbm.at[pl.ds(tok, tchunk), pl.ds(c * EMB, EMB)]
            )
            return carry

        lax.fori_loop(0, 2 * CAT, body, 0)

    return gather_k


_TB = 2048


def _matmul(cat_emb, xt, Wn_pad, bn, Wocat, Wonum, bo, s_dim, b_dim):
    n = cat_emb.shape[0]
    nb = b_dim // _TB

    def body(cat_ref, x_ref, wn_ref, bn_ref, wc_ref, wo_ref, bo_ref, out_ref):
        lane = lax.broadcasted_iota(jnp.int32, (_TB, CPAD), 1)
        cat = jnp.where(lane < CAT_EMB, cat_ref[...], 0.0)
        x2 = x_ref[0]
        num_emb = (
            jnp.dot(x2, wn_ref[...], preferred_element_type=jnp.float32)
            + bn_ref[...]
        )
        acc = jnp.dot(cat, wc_ref[...], preferred_element_type=jnp.float32)
        acc = acc + jnp.dot(num_emb, wo_ref[...], preferred_element_type=jnp.float32)
        out_ref[...] = acc + bo_ref[...]

    return pl.pallas_call(
        body,
        grid=(s_dim, nb),
        in_specs=[
            pl.BlockSpec((_TB, CPAD), lambda i, j: (i * nb + j, 0)),
            pl.BlockSpec((1, _TB, XDIM), lambda i, j: (i, j, 0)),
            pl.BlockSpec((XDIM, NUMD), lambda i, j: (0, 0)),
            pl.BlockSpec((1, NUMD), lambda i, j: (0, 0)),
            pl.BlockSpec((CPAD, FDIM), lambda i, j: (0, 0)),
            pl.BlockSpec((NUMD, FDIM), lambda i, j: (0, 0)),
            pl.BlockSpec((1, FDIM), lambda i, j: (0, 0)),
        ],
        out_specs=pl.BlockSpec((_TB, FDIM), lambda i, j: (i * nb + j, 0)),
        out_shape=jax.ShapeDtypeStruct((n, FDIM), jnp.float32),
    )(cat_emb, xt, Wn_pad, bn, Wocat, Wonum, bo)


def kernel(x, tables, Wn, bn, Wo, bo):
    b, s, _ = x.shape
    n = b * s
    xt = jnp.swapaxes(x, 0, 1)
    v = jnp.transpose(xt[..., :CAT].astype(jnp.int32), (2, 0, 1))
    c_base = (jnp.arange(CAT, dtype=jnp.int32) * (VP // _TBLK))[:, None, None]
    blk = c_base + (v >> 10)
    w = v & 1023
    idx_cm = ((blk << 10) + ((w & 127) << 3) + (w >> 7)).reshape(CAT, n)
    tbl_lin = _transpose_tables(jnp.swapaxes(tables, 1, 2)).reshape(
        CAT * VP, EMB
    )
    cat_emb = _make_gather(n)(idx_cm, tbl_lin)
    Wn_pad = jnp.concatenate([jnp.zeros((CAT, NUMD), jnp.float32), Wn], axis=0)
    Wo_pad = jnp.concatenate(
        [Wo[:CAT_EMB], jnp.zeros((CPAD - CAT_EMB, FDIM), jnp.float32)], axis=0
    )
    out2 = _matmul(
        cat_emb,
        xt,
        Wn_pad,
        bn.reshape(1, NUMD),
        Wo_pad,
        Wo[CAT_EMB:],
        bo.reshape(1, FDIM),
        s,
        b,
    )
    return jnp.swapaxes(out2.reshape(s, b, FDIM), 0, 1)


# speedup vs baseline: 3.4052x; 2.4947x over previous
import functools

import jax
import jax.numpy as jnp
from jax import lax
from jax.experimental import pallas as pl
from jax.experimental.pallas import tpu as pltpu
from jax.experimental.pallas import tpu_sc as plsc

CAT = 26
NUMD = 50
XDIM = CAT + NUMD
VOCAB = 100000
EMB = 16
FDIM = 128
CAT_EMB = CAT * EMB
NPLANE = 4
_VB = 4096
_NQ = 25
_PGRP = 4
VPAD = _NQ * _VB


def _transpose_tables(tables_2d):
    def body(a_ref, out_ref):
        out_ref[...] = jnp.transpose(a_ref[...], (1, 0))

    return pl.pallas_call(
        body,
        grid=(_PGRP, _NQ),
        in_specs=[pl.BlockSpec((8 * EMB, _VB), lambda p, q: (p, q))],
        out_specs=pl.BlockSpec((_VB, 8 * EMB), lambda p, q: (p * _NQ + q, 0)),
        out_shape=jax.ShapeDtypeStruct((_PGRP * _NQ * _VB, 8 * EMB), jnp.float32),
    )(tables_2d)


@functools.lru_cache(maxsize=None)
def _make_gather(n_tok: int):
    info = plsc.get_sparse_core_info()
    nw = info.num_cores * info.num_subcores
    tok_per_w = n_tok // nw
    assert tok_per_w * nw == n_tok
    tchunk = tok_per_w // 2
    mesh = plsc.VectorSubcoreMesh(core_axis_name="c", subcore_axis_name="s")

    @functools.partial(
        pl.kernel,
        mesh=mesh,
        compiler_params=pltpu.CompilerParams(use_tc_tiling_on_sc=False),
        out_type=jax.ShapeDtypeStruct((NPLANE, n_tok, FDIM), jnp.float32),
        scratch_types=[
            pltpu.VMEM((tchunk,), jnp.int32),
            pltpu.VMEM((tchunk, EMB), jnp.float32),
            pltpu.SemaphoreType.DMA,
        ],
    )
    def gather_k(idx_hbm, table_hbm, out_hbm, idx_v, rows_v, sem):
        wid = lax.axis_index("s") * info.num_cores + lax.axis_index("c")
        w_base = wid * tok_per_w

        def body(step, carry):
            c = step // 2
            tok = w_base + (step % 2) * tchunk
            pltpu.sync_copy(idx_hbm.at[c, pl.ds(tok, tchunk)], idx_v)
            pltpu.async_copy(table_hbm.at[idx_v], rows_v, sem).wait()
            pltpu.sync_copy(
                rows_v,
                out_hbm.at[c // 8, pl.ds(tok, tchunk), pl.ds((c % 8) * EMB, EMB)],
            )
            return carry

        lax.fori_loop(0, 2 * CAT, body, 0)

    return gather_k


_TB = 2048


def _matmul(planes3, xt, Wn_pad, bn, Wocat, Wonum, bo, s_dim, b_dim):
    n = planes3.shape[1]
    nb = b_dim // _TB

    def body(cp, x_ref, wn_ref, bn_ref, wc_ref, wo_ref, bo_ref, out_ref):
        lane = lax.broadcasted_iota(jnp.int32, (_TB, FDIM), 1)
        c3v = jnp.where(lane < CAT_EMB - 3 * FDIM, cp[3], 0.0)
        cat = jnp.concatenate([cp[0], cp[1], cp[2], c3v], axis=1)
        num_emb = (
            jnp.dot(x_ref[0], wn_ref[...], preferred_element_type=jnp.float32)
            + bn_ref[...]
        )
        acc = jnp.dot(cat, wc_ref[...], preferred_element_type=jnp.float32)
        acc = acc + jnp.dot(num_emb, wo_ref[...], preferred_element_type=jnp.float32)
        out_ref[...] = acc + bo_ref[...]

    return pl.pallas_call(
        body,
        grid=(s_dim, nb),
        in_specs=[
            pl.BlockSpec((NPLANE, _TB, FDIM), lambda i, j: (0, i * nb + j, 0)),
            pl.BlockSpec((1, _TB, XDIM), lambda i, j: (i, j, 0)),
            pl.BlockSpec((XDIM, NUMD), lambda i, j: (0, 0)),
            pl.BlockSpec((1, NUMD), lambda i, j: (0, 0)),
            pl.BlockSpec((NPLANE * FDIM, FDIM), lambda i, j: (0, 0)),
            pl.BlockSpec((NUMD, FDIM), lambda i, j: (0, 0)),
            pl.BlockSpec((1, FDIM), lambda i, j: (0, 0)),
        ],
        out_specs=pl.BlockSpec((_TB, FDIM), lambda i, j: (i * nb + j, 0)),
        out_shape=jax.ShapeDtypeStruct((n, FDIM), jnp.float32),
    )(planes3, xt, Wn_pad, bn, Wocat, Wonum, bo)


def kernel(x, tables, Wn, bn, Wo, bo):
    b, s, _ = x.shape
    n = b * s
    xt = jnp.swapaxes(x, 0, 1)
    v = jnp.transpose(xt[..., :CAT].astype(jnp.int32), (2, 0, 1))
    cols = jnp.arange(CAT, dtype=jnp.int32)
    base_c = ((cols >> 3) * (_NQ * _VB * 8) + (cols & 7))[:, None, None]
    idx_cm = (base_c + ((v >> 12) << 15) + ((v & (_VB - 1)) << 3)).reshape(
        CAT, n
    )
    tbl_lin = _transpose_tables(
        jnp.swapaxes(tables, 1, 2).reshape(CAT * EMB, VOCAB)
    ).reshape(_PGRP * _NQ * _VB * 8, EMB)
    planes3 = _make_gather(n)(idx_cm, tbl_lin)
    Wn_pad = jnp.concatenate([jnp.zeros((CAT, NUMD), jnp.float32), Wn], axis=0)
    Wo_pad = jnp.concatenate(
        [Wo[:CAT_EMB], jnp.zeros((NPLANE * FDIM - CAT_EMB, FDIM), jnp.float32)],
        axis=0,
    )
    out2 = _matmul(
        planes3,
        xt,
        Wn_pad,
        bn.reshape(1, NUMD),
        Wo_pad,
        Wo[CAT_EMB:],
        bo.reshape(1, FDIM),
        s,
        b,
    )
    return jnp.swapaxes(out2.reshape(s, b, FDIM), 0, 1)


# speedup vs baseline: 3.6567x; 1.0739x over previous
import functools

import jax
import jax.numpy as jnp
from jax import lax
from jax.experimental import pallas as pl
from jax.experimental.pallas import tpu as pltpu
from jax.experimental.pallas import tpu_sc as plsc

CAT = 26
NUMD = 50
XDIM = CAT + NUMD
VOCAB = 100000
EMB = 16
FDIM = 128
CAT_EMB = CAT * EMB
NPLANE = 4
_VB = 4096
_NQ = 25
_PGRP = 4
VPAD = _NQ * _VB


def _transpose_tables(tables_2d):
    def body(a_ref, out_ref):
        out_ref[...] = jnp.transpose(a_ref[...], (1, 0))

    return pl.pallas_call(
        body,
        grid=(_PGRP, _NQ),
        in_specs=[pl.BlockSpec((8 * EMB, _VB), lambda p, q: (p, q))],
        out_specs=pl.BlockSpec((_VB, 8 * EMB), lambda p, q: (p * _NQ + q, 0)),
        out_shape=jax.ShapeDtypeStruct((_PGRP * _NQ * _VB, 8 * EMB), jnp.float32),
    )(tables_2d)


@functools.lru_cache(maxsize=None)
def _make_gather(n_tok: int):
    info = plsc.get_sparse_core_info()
    nw = info.num_cores * info.num_subcores
    tok_per_w = n_tok // nw
    assert tok_per_w * nw == n_tok
    tchunk = tok_per_w // 2
    mesh = plsc.VectorSubcoreMesh(core_axis_name="c", subcore_axis_name="s")

    @functools.partial(
        pl.kernel,
        mesh=mesh,
        compiler_params=pltpu.CompilerParams(use_tc_tiling_on_sc=False),
        out_type=jax.ShapeDtypeStruct((NPLANE, n_tok, FDIM), jnp.float32),
        scratch_types=[
            pltpu.VMEM((2, tchunk), jnp.int32),
            pltpu.VMEM((2, tchunk, EMB), jnp.float32),
            pltpu.SemaphoreType.DMA,
            pltpu.SemaphoreType.DMA,
        ],
    )
    def gather_k(idx_hbm, table_hbm, out_hbm, idx_v, rows_v, gsem, wsem):
        wid = lax.axis_index("s") * info.num_cores + lax.axis_index("c")
        w_base = wid * tok_per_w
        nstep = 2 * CAT

        def idx_load(s):
            pltpu.sync_copy(
                idx_hbm.at[s // 2, pl.ds(w_base + (s % 2) * tchunk, tchunk)],
                idx_v.at[s % 2],
            )

        def out_slice(s):
            c = s // 2
            tok = w_base + (s % 2) * tchunk
            return out_hbm.at[
                c // 8, pl.ds(tok, tchunk), pl.ds((c % 8) * EMB, EMB)
            ]

        def gather_copy(s):
            return pltpu.make_async_copy(
                table_hbm.at[idx_v.at[s % 2]], rows_v.at[s % 2], gsem
            )

        def write_copy(s):
            return pltpu.make_async_copy(rows_v.at[s % 2], out_slice(s), wsem)

        idx_load(0)
        gather_copy(0).start()
        idx_load(1)

        def body(s, carry):
            gather_copy(s).wait()
            write_copy(s).start()

            @pl.when(s + 2 < nstep)
            def _():
                idx_load(s + 2)

            @pl.when(s + 1 < nstep)
            def _():
                @pl.when(s >= 1)
                def _():
                    write_copy(s - 1).wait()

                gather_copy(s + 1).start()

            return carry

        lax.fori_loop(0, nstep, body, 0)
        write_copy(nstep - 2).wait()
        write_copy(nstep - 1).wait()

    return gather_k


_TB = 2048


def _matmul(planes3, xt, Wn_pad, bn, Wocat, Wonum, bo, s_dim, b_dim):
    n = planes3.shape[1]
    nb = b_dim // _TB

    def body(cp, x_ref, wn_ref, bn_ref, wc_ref, wo_ref, bo_ref, out_ref):
        lane = lax.broadcasted_iota(jnp.int32, (_TB, FDIM), 1)
        c3v = jnp.where(lane < CAT_EMB - 3 * FDIM, cp[3], 0.0)
        cat = jnp.concatenate([cp[0], cp[1], cp[2], c3v], axis=1).astype(
            jnp.bfloat16
        )
        num_emb = (
            jnp.dot(x_ref[0], wn_ref[...], preferred_element_type=jnp.float32)
            + bn_ref[...]
        )
        acc = jnp.dot(cat, wc_ref[...], preferred_element_type=jnp.float32)
        acc = acc + jnp.dot(num_emb, wo_ref[...], preferred_element_type=jnp.float32)
        out_ref[...] = acc + bo_ref[...]

    return pl.pallas_call(
        body,
        grid=(s_dim, nb),
        in_specs=[
            pl.BlockSpec((NPLANE, _TB, FDIM), lambda i, j: (0, i * nb + j, 0)),
            pl.BlockSpec((1, _TB, XDIM), lambda i, j: (i, j, 0)),
            pl.BlockSpec((XDIM, NUMD), lambda i, j: (0, 0)),
            pl.BlockSpec((1, NUMD), lambda i, j: (0, 0)),
            pl.BlockSpec((NPLANE * FDIM, FDIM), lambda i, j: (0, 0)),
            pl.BlockSpec((NUMD, FDIM), lambda i, j: (0, 0)),
            pl.BlockSpec((1, FDIM), lambda i, j: (0, 0)),
        ],
        out_specs=pl.BlockSpec((_TB, FDIM), lambda i, j: (i * nb + j, 0)),
        out_shape=jax.ShapeDtypeStruct((n, FDIM), jnp.float32),
    )(planes3, xt, Wn_pad, bn, Wocat, Wonum, bo)


def kernel(x, tables, Wn, bn, Wo, bo):
    b, s, _ = x.shape
    n = b * s
    xt = jnp.swapaxes(x, 0, 1)
    v = jnp.transpose(xt[..., :CAT].astype(jnp.int32), (2, 0, 1))
    cols = jnp.arange(CAT, dtype=jnp.int32)
    base_c = ((cols >> 3) * (_NQ * _VB * 8) + (cols & 7))[:, None, None]
    idx_cm = (base_c + ((v >> 12) << 15) + ((v & (_VB - 1)) << 3)).reshape(
        CAT, n
    )
    tbl_lin = _transpose_tables(
        jnp.swapaxes(tables, 1, 2).reshape(CAT * EMB, VOCAB)
    ).reshape(_PGRP * _NQ * _VB * 8, EMB)
    planes3 = _make_gather(n)(idx_cm, tbl_lin)
    Wn_pad = jnp.concatenate([jnp.zeros((CAT, NUMD), jnp.float32), Wn], axis=0)
    Wo_pad = jnp.concatenate(
        [Wo[:CAT_EMB], jnp.zeros((NPLANE * FDIM - CAT_EMB, FDIM), jnp.float32)],
        axis=0,
    ).astype(jnp.bfloat16)
    out2 = _matmul(
        planes3,
        xt,
        Wn_pad,
        bn.reshape(1, NUMD),
        Wo_pad,
        Wo[CAT_EMB:],
        bo.reshape(1, FDIM),
        s,
        b,
    )
    return jnp.swapaxes(out2.reshape(s, b, FDIM), 0, 1)


# speedup vs baseline: 3.6866x; 1.0082x over previous
import functools

import jax
import jax.numpy as jnp
from jax import lax
from jax.experimental import pallas as pl
from jax.experimental.pallas import tpu as pltpu
from jax.experimental.pallas import tpu_sc as plsc

CAT = 26
NUMD = 50
XDIM = CAT + NUMD
VOCAB = 100000
EMB = 16
FDIM = 128
CAT_EMB = CAT * EMB
NPLANE = 4
_VB = 4096
_NQ = 25
_PGRP = 4
VPAD = _NQ * _VB


def _transpose_tables(tables_2d):
    def body(a_ref, out_ref):
        out_ref[...] = jnp.transpose(a_ref[...], (1, 0))

    return pl.pallas_call(
        body,
        grid=(_PGRP, _NQ),
        in_specs=[pl.BlockSpec((8 * EMB, _VB), lambda p, q: (p, q))],
        out_specs=pl.BlockSpec((_VB, 8 * EMB), lambda p, q: (p * _NQ + q, 0)),
        out_shape=jax.ShapeDtypeStruct((_PGRP * _NQ * _VB, 8 * EMB), jnp.float32),
    )(tables_2d)


@functools.lru_cache(maxsize=None)
def _make_gather(n_tok: int):
    info = plsc.get_sparse_core_info()
    nw = info.num_cores * info.num_subcores
    tok_per_w = n_tok // nw
    assert tok_per_w * nw == n_tok
    halves = 1 if tok_per_w <= 3200 else 2
    tchunk = tok_per_w // halves
    mesh = plsc.VectorSubcoreMesh(core_axis_name="c", subcore_axis_name="s")

    @functools.partial(
        pl.kernel,
        mesh=mesh,
        compiler_params=pltpu.CompilerParams(use_tc_tiling_on_sc=False),
        out_type=jax.ShapeDtypeStruct((NPLANE, n_tok, FDIM), jnp.float32),
        scratch_types=[
            pltpu.VMEM((2, tchunk), jnp.int32),
            pltpu.VMEM((2, tchunk, EMB), jnp.float32),
            pltpu.SemaphoreType.DMA,
            pltpu.SemaphoreType.DMA,
        ],
    )
    def gather_k(idx_hbm, table_hbm, out_hbm, idx_v, rows_v, gsem, wsem):
        wid = lax.axis_index("s") * info.num_cores + lax.axis_index("c")
        w_base = wid * tok_per_w
        nstep = halves * CAT

        def idx_load(s):
            pltpu.sync_copy(
                idx_hbm.at[
                    s // halves, pl.ds(w_base + (s % halves) * tchunk, tchunk)
                ],
                idx_v.at[s % 2],
            )

        def out_slice(s):
            c = s // halves
            tok = w_base + (s % halves) * tchunk
            return out_hbm.at[
                c // 8, pl.ds(tok, tchunk), pl.ds((c % 8) * EMB, EMB)
            ]

        def gather_copy(s):
            return pltpu.make_async_copy(
                table_hbm.at[idx_v.at[s % 2]], rows_v.at[s % 2], gsem
            )

        def write_copy(s):
            return pltpu.make_async_copy(rows_v.at[s % 2], out_slice(s), wsem)

        idx_load(0)
        gather_copy(0).start()
        idx_load(1)

        def body(s, carry):
            gather_copy(s).wait()
            write_copy(s).start()

            @pl.when(s + 2 < nstep)
            def _():
                idx_load(s + 2)

            @pl.when(s + 1 < nstep)
            def _():
                @pl.when(s >= 1)
                def _():
                    write_copy(s - 1).wait()

                gather_copy(s + 1).start()

            return carry

        lax.fori_loop(0, nstep, body, 0)
        write_copy(nstep - 2).wait()
        write_copy(nstep - 1).wait()

    return gather_k


_TB = 2048


def _matmul_chunk(
    planes3, xt, Wn_pad, bn, Wocat, Wonum, bo, out_prev, n, s0, s_chunk, b_dim
):
    nb = b_dim // _TB

    def body(cp, x_ref, wn_ref, bn_ref, wc_ref, wo_ref, bo_ref, prev, out_ref):
        del prev
        lane = lax.broadcasted_iota(jnp.int32, (_TB, FDIM), 1)
        c3v = jnp.where(lane < CAT_EMB - 3 * FDIM, cp[3], 0.0)
        cat = jnp.concatenate([cp[0], cp[1], cp[2], c3v], axis=1)
        num_emb = (
            jnp.dot(x_ref[0], wn_ref[...], preferred_element_type=jnp.float32)
            + bn_ref[...]
        )
        acc = jnp.dot(cat, wc_ref[...], preferred_element_type=jnp.float32)
        acc = acc + jnp.dot(num_emb, wo_ref[...], preferred_element_type=jnp.float32)
        out_ref[...] = acc + bo_ref[...]

    return pl.pallas_call(
        body,
        grid=(s_chunk, nb),
        in_specs=[
            pl.BlockSpec((NPLANE, _TB, FDIM), lambda i, j: (0, i * nb + j, 0)),
            pl.BlockSpec((1, _TB, XDIM), lambda i, j: (s0 + i, j, 0)),
            pl.BlockSpec((XDIM, NUMD), lambda i, j: (0, 0)),
            pl.BlockSpec((1, NUMD), lambda i, j: (0, 0)),
            pl.BlockSpec((NPLANE * FDIM, FDIM), lambda i, j: (0, 0)),
            pl.BlockSpec((NUMD, FDIM), lambda i, j: (0, 0)),
            pl.BlockSpec((1, FDIM), lambda i, j: (0, 0)),
            pl.BlockSpec(memory_space=pl.ANY),
        ],
        out_specs=pl.BlockSpec((_TB, FDIM), lambda i, j: ((s0 + i) * nb + j, 0)),
        out_shape=jax.ShapeDtypeStruct((n, FDIM), jnp.float32),
        input_output_aliases={7: 0},
    )(planes3, xt, Wn_pad, bn, Wocat, Wonum, bo, out_prev)


_K = 5


def kernel(x, tables, Wn, bn, Wo, bo):
    b, s, _ = x.shape
    n = b * s
    sk = s // _K
    nc = sk * b
    xt = jnp.swapaxes(x, 0, 1)
    cols = jnp.arange(CAT, dtype=jnp.int32)
    base_c = ((cols >> 3) * (_NQ * _VB * 8) + (cols & 7))[:, None, None]
    tbl_lin = _transpose_tables(
        jnp.swapaxes(tables, 1, 2).reshape(CAT * EMB, VOCAB)
    ).reshape(_PGRP * _NQ * _VB * 8, EMB)
    Wn_pad = jnp.concatenate([jnp.zeros((CAT, NUMD), jnp.float32), Wn], axis=0)
    Wo_pad = jnp.concatenate(
        [Wo[:CAT_EMB], jnp.zeros((NPLANE * FDIM - CAT_EMB, FDIM), jnp.float32)],
        axis=0,
    )
    bn2 = bn.reshape(1, NUMD)
    bo2 = bo.reshape(1, FDIM)
    outbuf = jnp.zeros((n, FDIM), jnp.float32)
    for k in range(_K):
        v = jnp.transpose(
            xt[k * sk : (k + 1) * sk, :, :CAT].astype(jnp.int32), (2, 0, 1)
        )
        idx_k = (base_c + ((v >> 12) << 15) + ((v & (_VB - 1)) << 3)).reshape(
            CAT, nc
        )
        planes_k = _make_gather(nc)(idx_k, tbl_lin)
        outbuf = _matmul_chunk(
            planes_k, xt, Wn_pad, bn2, Wo_pad, Wo[CAT_EMB:], bo2,
            outbuf, n, k * sk, sk, b,
        )
    return jnp.swapaxes(outbuf.reshape(s, b, FDIM), 0, 1)


# speedup vs baseline: 3.7320x; 1.0123x over previous
import functools

import jax
import jax.numpy as jnp
from jax import lax
from jax.experimental import pallas as pl
from jax.experimental.pallas import tpu as pltpu
from jax.experimental.pallas import tpu_sc as plsc

CAT = 26
NUMD = 50
XDIM = CAT + NUMD
VOCAB = 100000
EMB = 16
FDIM = 128
CAT_EMB = CAT * EMB
NPLANE = 4
_VB = 4096
_NQ = 25
_PGRP = 4
VPAD = _NQ * _VB


def _transpose_tables(tables_2d):
    def body(a_ref, out_ref):
        out_ref[...] = jnp.transpose(a_ref[...], (1, 0))

    return pl.pallas_call(
        body,
        grid=(_PGRP, _NQ),
        in_specs=[pl.BlockSpec((8 * EMB, _VB), lambda p, q: (p, q))],
        out_specs=pl.BlockSpec((_VB, 8 * EMB), lambda p, q: (p * _NQ + q, 0)),
        out_shape=jax.ShapeDtypeStruct((_PGRP * _NQ * _VB, 8 * EMB), jnp.float32),
    )(tables_2d)


@functools.lru_cache(maxsize=None)
def _make_gather(n_tok: int):
    info = plsc.get_sparse_core_info()
    nw = info.num_cores * info.num_subcores
    tok_per_w = n_tok // nw
    assert tok_per_w * nw == n_tok
    halves = 1 if tok_per_w <= 3200 else 2
    tchunk = tok_per_w // halves
    mesh = plsc.VectorSubcoreMesh(core_axis_name="c", subcore_axis_name="s")

    @functools.partial(
        pl.kernel,
        mesh=mesh,
        compiler_params=pltpu.CompilerParams(use_tc_tiling_on_sc=False),
        out_type=jax.ShapeDtypeStruct((NPLANE, n_tok, FDIM), jnp.float32),
        scratch_types=[
            pltpu.VMEM((2, tchunk), jnp.int32),
            pltpu.VMEM((2, tchunk, EMB), jnp.float32),
            pltpu.SemaphoreType.DMA,
            pltpu.SemaphoreType.DMA,
        ],
    )
    def gather_k(idx_hbm, table_hbm, out_hbm, idx_v, rows_v, gsem, wsem):
        wid = lax.axis_index("s") * info.num_cores + lax.axis_index("c")
        w_base = wid * tok_per_w
        nstep = halves * CAT

        def idx_load(s):
            pltpu.sync_copy(
                idx_hbm.at[
                    s // halves, pl.ds(w_base + (s % halves) * tchunk, tchunk)
                ],
                idx_v.at[s % 2],
            )

        def out_slice(s):
            c = s // halves
            tok = w_base + (s % halves) * tchunk
            return out_hbm.at[
                c // 8, pl.ds(tok, tchunk), pl.ds((c % 8) * EMB, EMB)
            ]

        def gather_copy(s):
            return pltpu.make_async_copy(
                table_hbm.at[idx_v.at[s % 2]], rows_v.at[s % 2], gsem
            )

        def write_copy(s):
            return pltpu.make_async_copy(rows_v.at[s % 2], out_slice(s), wsem)

        idx_load(0)
        gather_copy(0).start()
        idx_load(1)

        def body(s, carry):
            gather_copy(s).wait()
            write_copy(s).start()

            @pl.when(s + 2 < nstep)
            def _():
                idx_load(s + 2)

            @pl.when(s + 1 < nstep)
            def _():
                @pl.when(s >= 1)
                def _():
                    write_copy(s - 1).wait()

                gather_copy(s + 1).start()

            return carry

        lax.fori_loop(0, nstep, body, 0)
        write_copy(nstep - 2).wait()
        write_copy(nstep - 1).wait()

    return gather_k


_TB = 2048


def _matmul_chunk(
    planes3, xt, Wn_pad, bn, Wocat, Wonum, bo, out_prev, n, s0, s_chunk, b_dim
):
    nb = b_dim // _TB

    def body(cp, x_ref, wn_ref, bn_ref, wc_ref, wo_ref, bo_ref, prev, out_ref):
        del prev
        lane = lax.broadcasted_iota(jnp.int32, (_TB, FDIM), 1)
        c3v = jnp.where(lane < CAT_EMB - 3 * FDIM, cp[3], 0.0)
        cat = jnp.concatenate([cp[0], cp[1], cp[2], c3v], axis=1)
        num_emb = (
            jnp.dot(x_ref[0], wn_ref[...], preferred_element_type=jnp.float32)
            + bn_ref[...]
        )
        acc = jnp.dot(cat, wc_ref[...], preferred_element_type=jnp.float32)
        acc = acc + jnp.dot(num_emb, wo_ref[...], preferred_element_type=jnp.float32)
        out_ref[...] = acc + bo_ref[...]

    return pl.pallas_call(
        body,
        grid=(s_chunk, nb),
        in_specs=[
            pl.BlockSpec((NPLANE, _TB, FDIM), lambda i, j: (0, i * nb + j, 0)),
            pl.BlockSpec((1, _TB, XDIM), lambda i, j: (s0 + i, j, 0)),
            pl.BlockSpec((XDIM, NUMD), lambda i, j: (0, 0)),
            pl.BlockSpec((1, NUMD), lambda i, j: (0, 0)),
            pl.BlockSpec((NPLANE * FDIM, FDIM), lambda i, j: (0, 0)),
            pl.BlockSpec((NUMD, FDIM), lambda i, j: (0, 0)),
            pl.BlockSpec((1, FDIM), lambda i, j: (0, 0)),
            pl.BlockSpec(memory_space=pl.ANY),
        ],
        out_specs=pl.BlockSpec((_TB, FDIM), lambda i, j: ((s0 + i) * nb + j, 0)),
        out_shape=jax.ShapeDtypeStruct((n, FDIM), jnp.float32),
        input_output_aliases={7: 0},
    )(planes3, xt, Wn_pad, bn, Wocat, Wonum, bo, out_prev)


_K = 2


def kernel(x, tables, Wn, bn, Wo, bo):
    b, s, _ = x.shape
    n = b * s
    sk = s // _K
    nc = sk * b
    xt = jnp.swapaxes(x, 0, 1)
    cols = jnp.arange(CAT, dtype=jnp.int32)
    base_c = ((cols >> 3) * (_NQ * _VB * 8) + (cols & 7))[:, None, None]
    tbl_lin = _transpose_tables(
        jnp.swapaxes(tables, 1, 2).reshape(CAT * EMB, VOCAB)
    ).reshape(_PGRP * _NQ * _VB * 8, EMB)
    Wn_pad = jnp.concatenate([jnp.zeros((CAT, NUMD), jnp.float32), Wn], axis=0)
    Wo_pad = jnp.concatenate(
        [Wo[:CAT_EMB], jnp.zeros((NPLANE * FDIM - CAT_EMB, FDIM), jnp.float32)],
        axis=0,
    )
    bn2 = bn.reshape(1, NUMD)
    bo2 = bo.reshape(1, FDIM)
    outbuf = jnp.zeros((n, FDIM), jnp.float32)
    for k in range(_K):
        v = jnp.transpose(
            xt[k * sk : (k + 1) * sk, :, :CAT].astype(jnp.int32), (2, 0, 1)
        )
        idx_k = (base_c + ((v >> 12) << 15) + ((v & (_VB - 1)) << 3)).reshape(
            CAT, nc
        )
        planes_k = _make_gather(nc)(idx_k, tbl_lin)
        outbuf = _matmul_chunk(
            planes_k, xt, Wn_pad, bn2, Wo_pad, Wo[CAT_EMB:], bo2,
            outbuf, n, k * sk, sk, b,
        )
    return jnp.swapaxes(outbuf.reshape(s, b, FDIM), 0, 1)


# speedup vs baseline: 3.8382x; 1.0285x over previous
import functools

import jax
import jax.numpy as jnp
from jax import lax
from jax.experimental import pallas as pl
from jax.experimental.pallas import tpu as pltpu
from jax.experimental.pallas import tpu_sc as plsc

CAT = 26
NUMD = 50
XDIM = CAT + NUMD
VOCAB = 100000
EMB = 16
FDIM = 128
CAT_EMB = CAT * EMB
NPLANE = 4
_VB = 4096
_NQ = 25
_PGRP = 4
VPAD = _NQ * _VB


def _transpose_tables(tables_2d):
    def body(a_ref, out_ref):
        out_ref[...] = jnp.transpose(a_ref[...], (1, 0))

    return pl.pallas_call(
        body,
        grid=(_PGRP, _NQ),
        in_specs=[pl.BlockSpec((8 * EMB, _VB), lambda p, q: (p, q))],
        out_specs=pl.BlockSpec((_VB, 8 * EMB), lambda p, q: (p * _NQ + q, 0)),
        out_shape=jax.ShapeDtypeStruct((_PGRP * _NQ * _VB, 8 * EMB), jnp.float32),
    )(tables_2d)


@functools.lru_cache(maxsize=None)
def _make_gather(n_tok: int):
    info = plsc.get_sparse_core_info()
    nw = info.num_cores * info.num_subcores
    tok_per_w = n_tok // nw
    assert tok_per_w * nw == n_tok
    halves = 1 if tok_per_w <= 3200 else 2
    tchunk = tok_per_w // halves
    mesh = plsc.VectorSubcoreMesh(core_axis_name="c", subcore_axis_name="s")

    @functools.partial(
        pl.kernel,
        mesh=mesh,
        compiler_params=pltpu.CompilerParams(use_tc_tiling_on_sc=False),
        out_type=jax.ShapeDtypeStruct((NPLANE, n_tok, FDIM), jnp.float32),
        scratch_types=[
            pltpu.VMEM((2, tchunk), jnp.int32),
            pltpu.VMEM((2, tchunk, EMB), jnp.float32),
            pltpu.SemaphoreType.DMA,
            pltpu.SemaphoreType.DMA,
        ],
    )
    def gather_k(idx_hbm, table_hbm, out_hbm, idx_v, rows_v, gsem, wsem):
        wid = lax.axis_index("s") * info.num_cores + lax.axis_index("c")
        w_base = wid * tok_per_w
        nstep = halves * CAT

        def idx_load(s):
            pltpu.sync_copy(
                idx_hbm.at[
                    s // halves, pl.ds(w_base + (s % halves) * tchunk, tchunk)
                ],
                idx_v.at[s % 2],
            )

        def out_slice(s):
            c = s // halves
            tok = w_base + (s % halves) * tchunk
            return out_hbm.at[
                c // 8, pl.ds(tok, tchunk), pl.ds((c % 8) * EMB, EMB)
            ]

        def gather_copy(s):
            return pltpu.make_async_copy(
                table_hbm.at[idx_v.at[s % 2]], rows_v.at[s % 2], gsem
            )

        def write_copy(s):
            return pltpu.make_async_copy(rows_v.at[s % 2], out_slice(s), wsem)

        idx_load(0)
        gather_copy(0).start()
        idx_load(1)

        def body(s, carry):
            gather_copy(s).wait()
            write_copy(s).start()

            @pl.when(s + 2 < nstep)
            def _():
                idx_load(s + 2)

            @pl.when(s + 1 < nstep)
            def _():
                @pl.when(s >= 1)
                def _():
                    write_copy(s - 1).wait()

                gather_copy(s + 1).start()

            return carry

        lax.fori_loop(0, nstep, body, 0)
        write_copy(nstep - 2).wait()
        write_copy(nstep - 1).wait()

    return gather_k


_TB = 2048


def _matmul_chunk(
    planes3, xt, Wn_pad, bn, Wocat, Wonum, bo, out_prev, n, s0, s_chunk, b_dim
):
    nb = b_dim // _TB

    def body(cp, x_ref, wn_ref, bn_ref, wc_ref, wo_ref, bo_ref, *prev_out):
        out_ref = prev_out[-1]
        lane = lax.broadcasted_iota(jnp.int32, (_TB, FDIM), 1)
        c3v = jnp.where(lane < CAT_EMB - 3 * FDIM, cp[3], 0.0)
        cat = jnp.concatenate([cp[0], cp[1], cp[2], c3v], axis=1)
        num_emb = (
            jnp.dot(x_ref[0], wn_ref[...], preferred_element_type=jnp.float32)
            + bn_ref[...]
        )
        acc = jnp.dot(cat, wc_ref[...], preferred_element_type=jnp.float32)
        acc = acc + jnp.dot(num_emb, wo_ref[...], preferred_element_type=jnp.float32)
        out_ref[...] = acc + bo_ref[...]

    in_specs = [
        pl.BlockSpec((NPLANE, _TB, FDIM), lambda i, j: (0, i * nb + j, 0)),
        pl.BlockSpec((1, _TB, XDIM), lambda i, j: (s0 + i, j, 0)),
        pl.BlockSpec((XDIM, NUMD), lambda i, j: (0, 0)),
        pl.BlockSpec((1, NUMD), lambda i, j: (0, 0)),
        pl.BlockSpec((NPLANE * FDIM, FDIM), lambda i, j: (0, 0)),
        pl.BlockSpec((NUMD, FDIM), lambda i, j: (0, 0)),
        pl.BlockSpec((1, FDIM), lambda i, j: (0, 0)),
    ]
    args = [planes3, xt, Wn_pad, bn, Wocat, Wonum, bo]
    aliases = {}
    if out_prev is not None:
        in_specs.append(pl.BlockSpec(memory_space=pl.ANY))
        args.append(out_prev)
        aliases = {7: 0}
    return pl.pallas_call(
        body,
        grid=(s_chunk, nb),
        in_specs=in_specs,
        out_specs=pl.BlockSpec((_TB, FDIM), lambda i, j: ((s0 + i) * nb + j, 0)),
        out_shape=jax.ShapeDtypeStruct((n, FDIM), jnp.float32),
        input_output_aliases=aliases,
    )(*args)


_K = 2


def kernel(x, tables, Wn, bn, Wo, bo):
    b, s, _ = x.shape
    n = b * s
    sk = s // _K
    nc = sk * b
    xt = jnp.swapaxes(x, 0, 1)
    cols = jnp.arange(CAT, dtype=jnp.int32)
    base_c = ((cols >> 3) * (_NQ * _VB * 8) + (cols & 7))[:, None, None]
    tbl_lin = _transpose_tables(
        jnp.swapaxes(tables, 1, 2).reshape(CAT * EMB, VOCAB)
    ).reshape(_PGRP * _NQ * _VB * 8, EMB)
    Wn_pad = jnp.concatenate([jnp.zeros((CAT, NUMD), jnp.float32), Wn], axis=0)
    Wo_pad = jnp.concatenate(
        [Wo[:CAT_EMB], jnp.zeros((NPLANE * FDIM - CAT_EMB, FDIM), jnp.float32)],
        axis=0,
    )
    bn2 = bn.reshape(1, NUMD)
    bo2 = bo.reshape(1, FDIM)
    outbuf = None
    for k in range(_K):
        v = jnp.transpose(
            xt[k * sk : (k + 1) * sk, :, :CAT].astype(jnp.int32), (2, 0, 1)
        )
        idx_k = (base_c + ((v >> 12) << 15) + ((v & (_VB - 1)) << 3)).reshape(
            CAT, nc
        )
        planes_k = _make_gather(nc)(idx_k, tbl_lin)
        outbuf = _matmul_chunk(
            planes_k, xt, Wn_pad, bn2, Wo_pad, Wo[CAT_EMB:], bo2,
            outbuf, n, k * sk, sk, b,
        )
    return jnp.swapaxes(outbuf.reshape(s, b, FDIM), 0, 1)
